# Initial kernel scaffold; baseline (speedup 1.0000x reference)
import jax
import jax.numpy as jnp
from jax.experimental import pallas as pl


def kernel(x, batch_index, weight):
    raise NotImplementedError("write your pallas kernel here")


# speedup vs baseline: 16.6575x; 16.6575x over previous
import jax
import jax.numpy as jnp
from jax.experimental import pallas as pl
from jax.experimental.pallas import tpu as pltpu

S = 16


def _fused(bi_ref, x_ref, w_ref, out_ref, gc_ref, counts_ref):
    p = pl.program_id(0)
    j = pl.program_id(1)
    nb = pl.program_id(2)
    NB = pl.num_programs(2)
    B = x_ref.shape[1]
    C = x_ref.shape[2]

    bi = bi_ref[0]
    seg_iota = jax.lax.broadcasted_iota(jnp.int32, (S, B), 0)
    onehot_t = (seg_iota == bi).astype(jnp.float32)
    x2 = x_ref[0]

    @pl.when(p == 0)
    def _pass_a():
        @pl.when(jnp.logical_and(j == 0, nb == 0))
        def _():
            counts_ref[...] = jnp.zeros_like(counts_ref)

        @pl.when(j == 0)
        def _():
            cnt = jnp.sum(onehot_t, axis=1, keepdims=True)
            counts_ref[...] += jnp.broadcast_to(cnt, (S, C))

        @pl.when(nb == 0)
        def _():
            gc_ref[j] = jnp.zeros((S, C), jnp.float32)

        gc_ref[j] += jnp.dot(onehot_t, x2, preferred_element_type=jnp.float32)

        @pl.when(nb == NB - 1)
        def _():
            cnt = jnp.clip(counts_ref[...], 1.0, None)
            mean = gc_ref[j] / cnt
            gc_ref[j] = jnp.tanh(
                jnp.dot(mean, w_ref[...], preferred_element_type=jnp.float32))

    @pl.when(p == 1)
    def _pass_b():
        gcj = gc_ref[j]
        scores_t = jax.lax.dot_general(
            gcj, x2, (((1,), (1,)), ((), ())),
            preferred_element_type=jnp.float32)
        s_row = jnp.sum(scores_t * onehot_t, axis=0, keepdims=True)
        weighted = onehot_t * jax.nn.sigmoid(s_row)

        @pl.when(nb == 0)
        def _():
            out_ref[0] = jnp.zeros((S, C), jnp.float32)

        out_ref[0] += jnp.dot(weighted, x2, preferred_element_type=jnp.float32)

        @pl.when(nb == NB - 1)
        def _():
            cnt = jnp.clip(counts_ref[...], 1.0, None)
            out_ref[0] = out_ref[0] / cnt


def kernel(x, batch_index, weight):
    J, F, C = x.shape
    B = 4096
    NB = F // B
    bi = batch_index.astype(jnp.int32).reshape(NB, 1, B)
    return pl.pallas_call(
        _fused,
        grid=(2, J, NB),
        in_specs=[
            pl.BlockSpec((1, 1, B), lambda p, j, nb: (nb, 0, 0)),
            pl.BlockSpec((1, B, C), lambda p, j, nb: (j, nb, 0)),
            pl.BlockSpec((C, C), lambda p, j, nb: (0, 0)),
        ],
        out_specs=pl.BlockSpec((1, S, C), lambda p, j, nb: (j, 0, 0)),
        out_shape=jax.ShapeDtypeStruct((J, S, C), jnp.float32),
        scratch_shapes=[
            pltpu.VMEM((J, S, C), jnp.float32),
            pltpu.VMEM((S, C), jnp.float32),
        ],
    )(bi, x, weight)


# speedup vs baseline: 22.6541x; 1.3600x over previous
import jax
import jax.numpy as jnp
from jax.experimental import pallas as pl
from jax.experimental.pallas import tpu as pltpu

S = 16


def _fused(bi_ref, x_ref, w_ref, out_ref, gc_ref, counts_ref):
    p = pl.program_id(0)
    j = pl.program_id(1)
    nb = pl.program_id(2)
    NB = pl.num_programs(2)
    B = x_ref.shape[1]
    C = x_ref.shape[2]

    bi = bi_ref[0]
    seg_iota = jax.lax.broadcasted_iota(jnp.int32, (S, B), 0)
    onehot_t = (seg_iota == bi).astype(jnp.float32)
    x2 = x_ref[0]

    @pl.when(p == 0)
    def _pass_a():
        @pl.when(jnp.logical_and(j == 0, nb == 0))
        def _():
            counts_ref[...] = jnp.zeros_like(counts_ref)

        @pl.when(j == 0)
        def _():
            cnt = jnp.sum(onehot_t, axis=1, keepdims=True)
            counts_ref[...] += jnp.broadcast_to(cnt, (S, C))

        @pl.when(nb == 0)
        def _():
            gc_ref[j] = jnp.zeros((S, C), jnp.float32)

        gc_ref[j] += jnp.dot(onehot_t, x2, preferred_element_type=jnp.float32)

        @pl.when(nb == NB - 1)
        def _():
            cnt = jnp.clip(counts_ref[...], 1.0, None)
            mean = gc_ref[j] / cnt
            gc_ref[j] = jnp.tanh(
                jnp.dot(mean, w_ref[...], preferred_element_type=jnp.float32))

    @pl.when(p == 1)
    def _pass_b():
        gcj = gc_ref[j]
        scores_t = jax.lax.dot_general(
            gcj, x2, (((1,), (1,)), ((), ())),
            preferred_element_type=jnp.float32)
        s_row = jnp.sum(scores_t * onehot_t, axis=0, keepdims=True)
        weighted = onehot_t * jax.nn.sigmoid(s_row)

        @pl.when(nb == 0)
        def _():
            out_ref[0] = jnp.zeros((S, C), jnp.float32)

        out_ref[0] += jnp.dot(weighted, x2, preferred_element_type=jnp.float32)

        @pl.when(nb == NB - 1)
        def _():
            cnt = jnp.clip(counts_ref[...], 1.0, None)
            out_ref[0] = out_ref[0] / cnt


def kernel(x, batch_index, weight):
    J, F, C = x.shape
    B = 8192
    NB = F // B
    bi = batch_index.astype(jnp.int32).reshape(NB, 1, B)
    return pl.pallas_call(
        _fused,
        grid=(2, J, NB),
        in_specs=[
            pl.BlockSpec((1, 1, B), lambda p, j, nb: (nb, 0, 0)),
            pl.BlockSpec((1, B, C), lambda p, j, nb: (j, nb, 0)),
            pl.BlockSpec((C, C), lambda p, j, nb: (0, 0)),
        ],
        out_specs=pl.BlockSpec((1, S, C), lambda p, j, nb: (j, 0, 0)),
        out_shape=jax.ShapeDtypeStruct((J, S, C), jnp.float32),
        scratch_shapes=[
            pltpu.VMEM((J, S, C), jnp.float32),
            pltpu.VMEM((S, C), jnp.float32),
        ],
    )(bi, x, weight)


# speedup vs baseline: 27.3999x; 1.2095x over previous
import jax
import jax.numpy as jnp
from jax.experimental import pallas as pl
from jax.experimental.pallas import tpu as pltpu

S = 16


def _fused(bi_ref, x_ref, w_ref, out_ref, gc_ref, counts_ref):
    p = pl.program_id(0)
    j = pl.program_id(1)
    nb = pl.program_id(2)
    NB = pl.num_programs(2)
    B = x_ref.shape[1]
    C = x_ref.shape[2]

    bi = bi_ref[0]
    seg_iota = jax.lax.broadcasted_iota(jnp.int32, (S, B), 0)
    onehot_t = (seg_iota == bi).astype(jnp.float32)
    x2 = x_ref[0]

    @pl.when(p == 0)
    def _pass_a():
        @pl.when(jnp.logical_and(j == 0, nb == 0))
        def _():
            counts_ref[...] = jnp.zeros_like(counts_ref)

        @pl.when(j == 0)
        def _():
            cnt = jnp.sum(onehot_t, axis=1, keepdims=True)
            counts_ref[...] += jnp.broadcast_to(cnt, (S, C))

        @pl.when(nb == 0)
        def _():
            gc_ref[j] = jnp.zeros((S, C), jnp.float32)

        gc_ref[j] += jnp.dot(onehot_t, x2, preferred_element_type=jnp.float32)

        @pl.when(nb == NB - 1)
        def _():
            cnt = jnp.clip(counts_ref[...], 1.0, None)
            mean = gc_ref[j] / cnt
            gc_ref[j] = jnp.tanh(
                jnp.dot(mean, w_ref[...], preferred_element_type=jnp.float32))

    @pl.when(p == 1)
    def _pass_b():
        gcj = gc_ref[j]
        scores_t = jax.lax.dot_general(
            gcj, x2, (((1,), (1,)), ((), ())),
            preferred_element_type=jnp.float32)
        s_row = jnp.sum(scores_t * onehot_t, axis=0, keepdims=True)
        weighted = onehot_t * jax.nn.sigmoid(s_row)

        @pl.when(nb == 0)
        def _():
            out_ref[0] = jnp.zeros((S, C), jnp.float32)

        out_ref[0] += jnp.dot(weighted, x2, preferred_element_type=jnp.float32)

        @pl.when(nb == NB - 1)
        def _():
            cnt = jnp.clip(counts_ref[...], 1.0, None)
            out_ref[0] = out_ref[0] / cnt


def kernel(x, batch_index, weight):
    J, F, C = x.shape
    B = 16384
    NB = F // B
    bi = batch_index.astype(jnp.int32).reshape(NB, 1, B)
    return pl.pallas_call(
        _fused,
        grid=(2, J, NB),
        in_specs=[
            pl.BlockSpec((1, 1, B), lambda p, j, nb: (nb, 0, 0)),
            pl.BlockSpec((1, B, C), lambda p, j, nb: (j, nb, 0)),
            pl.BlockSpec((C, C), lambda p, j, nb: (0, 0)),
        ],
        out_specs=pl.BlockSpec((1, S, C), lambda p, j, nb: (j, 0, 0)),
        out_shape=jax.ShapeDtypeStruct((J, S, C), jnp.float32),
        scratch_shapes=[
            pltpu.VMEM((J, S, C), jnp.float32),
            pltpu.VMEM((S, C), jnp.float32),
        ],
    )(bi, x, weight)
